# baseline (speedup 1.0000x reference)
import functools

import jax
import jax.numpy as jnp
from jax import lax
from jax.experimental import pallas as pl
from jax.experimental.pallas import tpu as pltpu
from jax.experimental.pallas import tpu_sc as plsc

N_NODES = 10000
N_EDGES = 320000
D = 128

NC = 2
NS = 16
NW = NC * NS
L = 16

E_C = 128
N_CHUNKS = N_EDGES // E_C
N_FULL = N_CHUNKS // NW
N_LEFT = N_CHUNKS - N_FULL * NW
NB = 3
N_STEADY = (N_FULL - 6) // NB
R_S = 624
R_REM = N_NODES - NS * R_S


def _sc_segment_sum(embed, edge_index, att):
    mesh = plsc.VectorSubcoreMesh(core_axis_name="c", subcore_axis_name="s")

    @functools.partial(
        pl.kernel,
        out_type=jax.ShapeDtypeStruct((NC, N_NODES, D), jnp.float32),
        mesh=mesh,
        scratch_types=[
            pltpu.VMEM((NB, 2, E_C), jnp.int32),
            pltpu.VMEM((NB, E_C), jnp.float32),
            pltpu.VMEM((NB, E_C, D), jnp.float32),
            pltpu.VMEM_SHARED((N_NODES, D), jnp.float32),
            [pltpu.SemaphoreType.DMA] * NB,
            [pltpu.SemaphoreType.DMA] * NB,
        ],
    )
    def k(embed_hbm, ei_hbm, att_hbm, out_hbm,
          ei_v, att_v, rows_v, acc, gsems, ssems):
        cid = lax.axis_index("c")
        sid = lax.axis_index("s")
        wid = sid * NC + cid

        zero = jnp.zeros((L,), jnp.float32)
        izero = jnp.zeros((L,), jnp.int32)

        for b0 in range(NB):
            for r0 in range(2):
                for j0 in range(E_C // L):
                    ei_v[b0, r0, pl.ds(j0 * L, L)] = izero

        def zero_row(r, _):
            for j in range(D // L):
                rows_v[0, r, pl.ds(j * L, L)] = zero
            return _

        lax.fori_loop(0, E_C, zero_row, None)
        row0 = sid * R_S
        for t in range(R_S // E_C):
            pltpu.sync_copy(rows_v.at[0], acc.at[pl.ds(row0 + t * E_C, E_C)])
        rem = R_S - (R_S // E_C) * E_C
        if rem:
            pltpu.sync_copy(rows_v.at[0, pl.ds(0, rem)],
                            acc.at[pl.ds(row0 + (R_S // E_C) * E_C, rem)])

        @pl.when(sid == NS - 1)
        def _():
            pltpu.sync_copy(rows_v.at[0, pl.ds(0, R_REM)],
                            acc.at[pl.ds(NS * R_S, R_REM)])

        plsc.subcore_barrier()

        def scale(b):
            def scale_block(kk, _):
                att16 = att_v[b, pl.ds(kk * L, L)]
                for l in range(L):
                    a = att16[l]
                    for j in range(D // L):
                        sl = pl.ds(j * L, L)
                        rows_v[b, kk * L + l, sl] = rows_v[b, kk * L + l, sl] * a
                return _

            lax.fori_loop(0, E_C // L, scale_block, None)

        def ebase(c):
            return (c * NW + wid) * E_C

        def gather(c, b):
            base = ebase(c)
            pltpu.sync_copy(ei_hbm.at[pl.ds(0, 2), pl.ds(base, E_C)],
                            ei_v.at[b])
            pltpu.async_copy(att_hbm.at[pl.ds(base, E_C)], att_v.at[b], gsems[b])
            pltpu.async_copy(embed_hbm.at[ei_v.at[b, 0]], rows_v.at[b], gsems[b])

        def wait_gather(c, b):
            base = ebase(c)
            pltpu.make_async_copy(att_hbm.at[pl.ds(base, E_C)], att_v.at[b],
                                  gsems[b]).wait()
            pltpu.make_async_copy(embed_hbm.at[ei_v.at[b, 0]], rows_v.at[b],
                                  gsems[b]).wait()

        def scatter(b):
            pltpu.async_copy(rows_v.at[b], acc.at[ei_v.at[b, 1]], ssems[b],
                             add=True)

        def wait_scatter(b):
            pltpu.make_async_copy(rows_v.at[b], acc.at[ei_v.at[b, 1]],
                                  ssems[b]).wait()

        def proc(c, b, nxt, wait_prev_scatter, issue_next):
            wait_gather(c, b)
            scale(b)
            scatter(b)
            if issue_next:
                if wait_prev_scatter:
                    wait_scatter(nxt)
                gather(c + 2, nxt)


        gather(0, 0)
        gather(1, 1)
        proc(0, 0, 2, False, True)
        proc(1, 1, 0, True, True)
        proc(2, 2, 1, True, True)

        def steady(t, _):
            c = NB * t + NB
            for u in range(NB):
                b = u
                nxt = (u + 2) % NB
                cc = c + u
                wait_gather(cc, b)
                scale(b)
                scatter(b)
                wait_scatter(nxt)
                gather(cc + 2, nxt)
            return _

        lax.fori_loop(0, N_STEADY, steady, None)

        proc(N_FULL - 3, (N_FULL - 3) % NB, (N_FULL - 1) % NB, True, True)
        proc(N_FULL - 2, (N_FULL - 2) % NB, 0, False, False)
        proc(N_FULL - 1, (N_FULL - 1) % NB, 0, False, False)

        @pl.when(wid < N_LEFT)
        def _():
            base = (N_FULL * NW + wid) * E_C
            wait_scatter(0)
            pltpu.sync_copy(ei_hbm.at[pl.ds(0, 2), pl.ds(base, E_C)],
                            ei_v.at[0])
            pltpu.async_copy(att_hbm.at[pl.ds(base, E_C)], att_v.at[0],
                             gsems[0])
            pltpu.async_copy(embed_hbm.at[ei_v.at[0, 0]], rows_v.at[0],
                             gsems[0])
            pltpu.make_async_copy(att_hbm.at[pl.ds(base, E_C)], att_v.at[0],
                                  gsems[0]).wait()
            pltpu.make_async_copy(embed_hbm.at[ei_v.at[0, 0]], rows_v.at[0],
                                  gsems[0]).wait()
            scale(0)
            scatter(0)

        for b in range(NB):
            wait_scatter(b)

        plsc.subcore_barrier()
        pltpu.sync_copy(acc.at[pl.ds(row0, R_S)],
                        out_hbm.at[cid, pl.ds(row0, R_S)])

        @pl.when(sid == NS - 1)
        def _():
            pltpu.sync_copy(acc.at[pl.ds(NS * R_S, R_REM)],
                            out_hbm.at[cid, pl.ds(NS * R_S, R_REM)])

    return k(embed, edge_index, att)


def _tc_tail_body(e_ref, p_ref, w_ref, b_ref, o_ref):
    h = e_ref[...] + p_ref[0] + p_ref[1]
    y = lax.dot_general(h, w_ref[...], (((1,), (1,)), ((), ())),
                        preferred_element_type=jnp.float32)
    y = y + b_ref[...]
    o_ref[...] = jnp.where(y >= 0, y, 0.01 * y)


def _tc_tail(embed, partials, W_w, W_b):
    BR = 1000
    grid = N_NODES // BR
    return pl.pallas_call(
        _tc_tail_body,
        grid=(grid,),
        in_specs=[
            pl.BlockSpec((BR, D), lambda i: (i, 0)),
            pl.BlockSpec((NC, BR, D), lambda i: (0, i, 0)),
            pl.BlockSpec((D, D), lambda i: (0, 0)),
            pl.BlockSpec((1, D), lambda i: (0, 0)),
        ],
        out_specs=pl.BlockSpec((BR, D), lambda i: (i, 0)),
        out_shape=jax.ShapeDtypeStruct((N_NODES, D), jnp.float32),
    )(embed, partials, W_w, W_b)


@jax.jit
def kernel(entity_embed, edge_index, edge_att, W_w, W_b):
    ei = edge_index.astype(jnp.int32)
    partials = _sc_segment_sum(entity_embed, ei, edge_att)
    return _tc_tail(entity_embed, partials, W_w, W_b.reshape(1, D))
